# Initial kernel scaffold; baseline (speedup 1.0000x reference)
import jax
import jax.numpy as jnp
from jax.experimental import pallas as pl


def kernel(batch_radicalindices, rademb_weight):
    raise NotImplementedError("write your pallas kernel here")


# speedup vs baseline: 1.0474x; 1.0474x over previous
import functools

import jax
import jax.numpy as jnp
from jax import lax
from jax.experimental import pallas as pl
from jax.experimental.pallas import tpu as pltpu
from jax.experimental.pallas import tpu_sc as plsc

BATCH = 4096
EMB_DIM = 128
MAX_NORM = 1.0
EPS = 1e-7

NUM_CORES = 2
NUM_SUBCORES = 16
LANES = 16
NUM_WORKERS = NUM_CORES * NUM_SUBCORES
B_PER_W = BATCH // NUM_WORKERS
CHUNKS = EMB_DIM // LANES
GROUP = LANES
N_GROUPS = B_PER_W // GROUP


def _sc_lookup(idx, table):
  mesh = plsc.VectorSubcoreMesh(core_axis_name="c", subcore_axis_name="s")

  @functools.partial(
      pl.kernel,
      mesh=mesh,
      out_type=jax.ShapeDtypeStruct((BATCH, EMB_DIM), jnp.float32),
      scratch_types=[
          pltpu.VMEM((B_PER_W,), jnp.int32),
          pltpu.VMEM((B_PER_W, EMB_DIM), jnp.float32),
          pltpu.VMEM((GROUP, LANES), jnp.float32),
          pltpu.VMEM((GROUP,), jnp.float32),
          pltpu.SemaphoreType.DMA,
      ],
      compiler_params=pltpu.CompilerParams(needs_layout_passes=False),
  )
  def body(idx_hbm, table_hbm, out_hbm, idx_v, rows_v, ss_mat, scale_v, sem):
    wid = lax.axis_index("s") * NUM_CORES + lax.axis_index("c")
    base = wid * B_PER_W
    pltpu.sync_copy(idx_hbm.at[pl.ds(base, B_PER_W)], idx_v)
    pltpu.async_copy(table_hbm.at[idx_v], rows_v, sem).wait()

    iota = lax.iota(jnp.int32, LANES)

    def group_fn(g, carry):
      r0 = g * GROUP
      for j in range(GROUP):
        acc = jnp.zeros((LANES,), jnp.float32)
        for c in range(CHUNKS):
          v = rows_v[r0 + j, pl.ds(c * LANES, LANES)]
          acc = acc + v * v
        ss_mat[j, :] = acc
      tot = jnp.zeros((LANES,), jnp.float32)
      for l in range(LANES):
        col = plsc.load_gather(ss_mat, [iota, jnp.full((LANES,), l, jnp.int32)])
        tot = tot + col
      tot = jnp.maximum(tot, 1e-30)
      i = lax.bitcast_convert_type(tot, jnp.int32)
      i = 0x5F3759DF - lax.shift_right_logical(i, 1)
      y = lax.bitcast_convert_type(i, jnp.float32)
      half = 0.5 * tot
      for _ in range(3):
        y = y * (1.5 - half * y * y)
      norm = tot * y
      scale = jnp.where(norm > MAX_NORM, MAX_NORM / (norm + EPS),
                        jnp.float32(1.0))
      for j in range(GROUP):
        splat = jnp.full((LANES,), scale[j], jnp.float32)
        for c in range(CHUNKS):
          rows_v[r0 + j, pl.ds(c * LANES, LANES)] = (
              rows_v[r0 + j, pl.ds(c * LANES, LANES)] * splat)
      return carry

    lax.fori_loop(0, N_GROUPS, group_fn, 0)
    pltpu.sync_copy(rows_v, out_hbm.at[pl.ds(base, B_PER_W)])

  return body(idx, table)


def kernel(batch_radicalindices, rademb_weight):
  idx = batch_radicalindices.reshape(-1).astype(jnp.int32)
  out = _sc_lookup(idx, rademb_weight)
  return out.reshape(BATCH, 1, EMB_DIM)
